# baseline (speedup 1.0000x reference)
import functools

import jax
import jax.numpy as jnp
from jax import lax
from jax.experimental import pallas as pl
from jax.experimental.pallas import tpu as pltpu
from jax.experimental.pallas import tpu_sc as plsc

T = 2048
D = 768
FF = 2048
NE = 8
TM = 256
NT = T * 2 // TM + (NE - 1)
XS = NT * TM
TF = 512
NF = FF // TF
CH = 256


def _router_body(x_ref, rw_ref, logits_ref, pm_ref, wm_ref, g_ref):
    x = x_ref[...]
    rw = rw_ref[...]
    logits = lax.dot_general(x, rw, (((1,), (1,)), ((), ())),
                             preferred_element_type=jnp.float32)
    logits_ref[...] = logits

    mx = jnp.max(logits, axis=1, keepdims=True)
    ex = jnp.exp(logits - mx)
    sm = ex / jnp.sum(ex, axis=1, keepdims=True)

    iota_e = lax.broadcasted_iota(jnp.int32, (T, NE), 1).astype(jnp.float32)
    m0 = jnp.max(logits, axis=1, keepdims=True)
    is0 = logits >= m0
    e0 = jnp.min(jnp.where(is0, iota_e, jnp.float32(NE)), axis=1, keepdims=True)
    oh0 = (iota_e == e0).astype(jnp.float32)
    masked = jnp.where(oh0 > 0, -jnp.inf, logits)
    m1 = jnp.max(masked, axis=1, keepdims=True)
    is1 = masked >= m1
    e1 = jnp.min(jnp.where(is1, iota_e, jnp.float32(NE)), axis=1, keepdims=True)
    oh1 = (iota_e == e1).astype(jnp.float32)

    p0v = jnp.sum(sm * oh0, axis=1, keepdims=True)
    p1v = jnp.sum(sm * oh1, axis=1, keepdims=True)
    den = p0v + p1v
    wm_ref[:, 0:1] = p0v / den
    wm_ref[:, 1:2] = p1v / den

    s = oh0 + oh1
    r = lax.broadcasted_iota(jnp.int32, (CH, CH), 0)
    c = lax.broadcasted_iota(jnp.int32, (CH, CH), 1)
    tri = (c < r).astype(jnp.float32)
    carry = jnp.zeros((1, NE), jnp.float32)
    chunks = []
    for k in range(T // CH):
        sc = lax.slice_in_dim(s, k * CH, (k + 1) * CH, axis=0)
        cc = lax.dot_general(tri, sc, (((1,), (0,)), ((), ())),
                             preferred_element_type=jnp.float32) + carry
        chunks.append(cc)
        carry = carry + jnp.sum(sc, axis=0, keepdims=True)
    cexc = jnp.concatenate(chunks, axis=0)
    counts = carry

    pc = jnp.ceil(counts / TM) * TM
    rr = lax.broadcasted_iota(jnp.int32, (NE, NE), 0)
    cc2 = lax.broadcasted_iota(jnp.int32, (NE, NE), 1)
    triu = (rr < cc2).astype(jnp.float32)
    pstart = lax.dot_general(pc, triu, (((1,), (0,)), ((), ())),
                             preferred_element_type=jnp.float32)

    rank0 = jnp.sum(cexc * oh0, axis=1, keepdims=True)
    rank1 = jnp.sum((cexc + oh0) * oh1, axis=1, keepdims=True)
    ps0 = jnp.sum(pstart * oh0, axis=1, keepdims=True)
    ps1 = jnp.sum(pstart * oh1, axis=1, keepdims=True)
    pm_ref[:, 0:1] = (ps0 + rank0).astype(jnp.int32)
    pm_ref[:, 1:2] = (ps1 + rank1).astype(jnp.int32)

    ti = lax.broadcasted_iota(jnp.int32, (32, 1), 0).astype(jnp.float32) * TM
    tib = jnp.broadcast_to(ti, (32, NE))
    psb = jnp.broadcast_to(pstart, (32, NE))
    nonempty = jnp.broadcast_to(pc > 0, (32, NE))
    cmp = (psb <= tib).astype(jnp.int32)
    g_raw = jnp.sum(cmp, axis=1, keepdims=True) - 1
    iota8 = lax.broadcasted_iota(jnp.int32, (1, NE), 1)
    last_e = jnp.max(jnp.where(counts > 0, iota8, 0), axis=1, keepdims=True)
    ntiles = (jnp.sum(pc, axis=1, keepdims=True) / TM).astype(jnp.int32)
    ntb = jnp.broadcast_to(ntiles, (32, 1))
    row_i = lax.broadcasted_iota(jnp.int32, (32, 1), 0)
    real = (row_i < ntb).astype(jnp.int32)
    started = ((psb <= tib) & nonempty).astype(jnp.int32)
    ordp1 = jnp.sum(started, axis=1, keepdims=True)
    iota8b = jnp.broadcast_to(iota8, (32, NE))
    nxt = jnp.min(jnp.where((psb > tib) & nonempty, iota8b, NE),
                  axis=1, keepdims=True)
    has_next = (nxt < NE).astype(jnp.int32) * real
    first = jnp.sum(((psb == tib) & nonempty).astype(jnp.int32),
                    axis=1, keepdims=True) * real
    g_ref[:, 0:1] = jnp.minimum(g_raw, last_e)
    g_ref[:, 1:2] = (ordp1 - 1) % 2
    g_ref[:, 2:3] = first
    g_ref[:, 3:4] = has_next
    g_ref[:, 4:5] = jnp.where(has_next > 0, nxt, 0)
    g_ref[:, 5:6] = ntb
    g_ref[:, 6:8] = jnp.zeros((32, 2), jnp.int32)


def _run_router(x, router_w):
    return pl.pallas_call(
        _router_body,
        out_shape=(
            jax.ShapeDtypeStruct((T, NE), jnp.float32),
            jax.ShapeDtypeStruct((T, 2), jnp.int32),
            jax.ShapeDtypeStruct((T, 2), jnp.float32),
            jax.ShapeDtypeStruct((32, 8), jnp.int32),
        ),
    )(x, router_w)


def _mlp_body(m_ref, x_ref, w1_hbm, w3_hbm, w2_hbm, out_ref,
              w1b, w3b, w2b, sems):
    i = pl.program_id(0)
    nt = m_ref[0, 5]
    slot = m_ref[i, 1]
    first = m_ref[i, 2]
    has_next = m_ref[i, 3]
    nextg = m_ref[i, 4]
    pairs = ((w1_hbm, w1b), (w3_hbm, w3b), (w2_hbm, w2b))

    def issue(e, s):
        for k, (hbm, buf) in enumerate(pairs):
            pltpu.make_async_copy(hbm.at[e], buf.at[s], sems.at[k, s]).start()

    def wait_one(k, s):
        hbm, buf = pairs[k]
        pltpu.make_async_copy(hbm.at[0], buf.at[s], sems.at[k, s]).wait()

    @pl.when(i == 0)
    def _():
        issue(m_ref[0, 0], 0)

    @pl.when((first == 1) & (has_next == 1) & (slot == 0))
    def _():
        issue(nextg, 1)

    @pl.when((first == 1) & (has_next == 1) & (slot == 1))
    def _():
        issue(nextg, 0)

    @pl.when((first == 1) & (slot == 0))
    def _():
        for k in range(3):
            wait_one(k, 0)

    @pl.when((first == 1) & (slot == 1))
    def _():
        for k in range(3):
            wait_one(k, 1)

    def compute(s):
        x = x_ref[...].astype(jnp.bfloat16)
        hf = FF // 2
        parts = []
        for h in range(2):
            fs = pl.ds(h * hf, hf)
            h1 = lax.dot_general(x, w1b[s, fs, :].astype(jnp.bfloat16),
                                 (((1,), (1,)), ((), ())),
                                 preferred_element_type=jnp.float32)
            h3 = lax.dot_general(x, w3b[s, fs, :].astype(jnp.bfloat16),
                                 (((1,), (1,)), ((), ())),
                                 preferred_element_type=jnp.float32)
            act = (h1 * jax.nn.sigmoid(h1) * h3).astype(jnp.bfloat16)
            parts.append(
                lax.dot_general(act, w2b[s, :, fs].astype(jnp.bfloat16),
                                (((1,), (1,)), ((), ())),
                                preferred_element_type=jnp.float32))
        out_ref[...] = parts[0] + parts[1]

    @pl.when((i < nt) & (slot == 0))
    def _():
        compute(0)

    @pl.when((i < nt) & (slot == 1))
    def _():
        compute(1)


def _run_mlp(x_sorted, w1, w3, w2, tile_meta):
    grid_spec = pltpu.PrefetchScalarGridSpec(
        num_scalar_prefetch=1,
        grid=(NT,),
        in_specs=[
            pl.BlockSpec((TM, D), lambda i, m: (i, 0)),
            pl.BlockSpec(memory_space=pl.ANY),
            pl.BlockSpec(memory_space=pl.ANY),
            pl.BlockSpec(memory_space=pl.ANY),
        ],
        out_specs=pl.BlockSpec((TM, D), lambda i, m: (i, 0)),
        scratch_shapes=[
            pltpu.VMEM((2, FF, D), jnp.float32),
            pltpu.VMEM((2, FF, D), jnp.float32),
            pltpu.VMEM((2, D, FF), jnp.float32),
            pltpu.SemaphoreType.DMA((3, 2)),
        ],
    )
    return pl.pallas_call(
        _mlp_body,
        grid_spec=grid_spec,
        out_shape=jax.ShapeDtypeStruct((XS, D), jnp.float32),
        compiler_params=pltpu.CompilerParams(
            dimension_semantics=("arbitrary",)),
    )(tile_meta, x_sorted, w1, w3, w2)


def _make_scatter():
    info = plsc.get_sparse_core_info()
    nw = info.num_cores * info.num_subcores
    tpw = T // nw
    mesh = plsc.VectorSubcoreMesh(core_axis_name="c", subcore_axis_name="s")

    @functools.partial(
        pl.kernel, mesh=mesh,
        out_type=jax.ShapeDtypeStruct((XS, D), jnp.float32),
        scratch_types=[
            pltpu.VMEM((tpw, D), jnp.float32),
            pltpu.VMEM((tpw,), jnp.int32),
            pltpu.SemaphoreType.DMA,
        ],
    )
    def scatter_k(x_hbm, p0_hbm, p1_hbm, xs_hbm, rows_v, idx_v, sem):
        wid = lax.axis_index("s") * info.num_cores + lax.axis_index("c")
        base = wid * tpw
        pltpu.sync_copy(x_hbm.at[pl.ds(base, tpw)], rows_v)
        pltpu.sync_copy(p0_hbm.at[pl.ds(base, tpw)], idx_v)
        pltpu.async_copy(rows_v, xs_hbm.at[idx_v], sem).wait()
        pltpu.sync_copy(p1_hbm.at[pl.ds(base, tpw)], idx_v)
        pltpu.async_copy(rows_v, xs_hbm.at[idx_v], sem).wait()

    return scatter_k


def _make_gather_add():
    info = plsc.get_sparse_core_info()
    nw = info.num_cores * info.num_subcores
    tpw = T // nw
    mesh = plsc.VectorSubcoreMesh(core_axis_name="c", subcore_axis_name="s")

    @functools.partial(
        pl.kernel, mesh=mesh,
        out_type=jax.ShapeDtypeStruct((T, D), jnp.float32),
        scratch_types=[
            pltpu.VMEM((tpw, D), jnp.float32),
            pltpu.VMEM((tpw, D), jnp.float32),
            pltpu.VMEM((tpw,), jnp.int32),
            pltpu.VMEM((tpw,), jnp.float32),
            pltpu.VMEM((tpw,), jnp.float32),
            pltpu.SemaphoreType.DMA,
        ],
    )
    def gather_k(ys_hbm, p0_hbm, p1_hbm, w0_hbm, w1_hbm, out_hbm,
                 r0, r1, idx_v, wv0, wv1, sem):
        wid = lax.axis_index("s") * info.num_cores + lax.axis_index("c")
        base = wid * tpw
        pltpu.sync_copy(p0_hbm.at[pl.ds(base, tpw)], idx_v)
        pltpu.async_copy(ys_hbm.at[idx_v], r0, sem).wait()
        pltpu.sync_copy(p1_hbm.at[pl.ds(base, tpw)], idx_v)
        pltpu.async_copy(ys_hbm.at[idx_v], r1, sem).wait()
        pltpu.sync_copy(w0_hbm.at[pl.ds(base, tpw)], wv0)
        pltpu.sync_copy(w1_hbm.at[pl.ds(base, tpw)], wv1)

        dn = lax.GatherDimensionNumbers(offset_dims=(),
                                        collapsed_slice_dims=(0,),
                                        start_index_map=(0,))

        def row_fma(jt, carry):
            cb = pl.multiple_of((jt // 16) * 16, 8)
            ln = jnp.full((16, 1), jt % 16, jnp.int32)
            ch0 = wv0[pl.ds(cb, 16)]
            ch1 = wv1[pl.ds(cb, 16)]
            b0 = lax.gather(ch0, ln, dn, (1,),
                            mode=lax.GatherScatterMode.PROMISE_IN_BOUNDS)
            b1 = lax.gather(ch1, ln, dn, (1,),
                            mode=lax.GatherScatterMode.PROMISE_IN_BOUNDS)
            for c in range(D // 16):
                sl = pl.ds(c * 16, 16)
                r0[jt, sl] = r0[jt, sl] * b0 + r1[jt, sl] * b1
            return carry

        lax.fori_loop(0, tpw, row_fma, 0)
        pltpu.sync_copy(r0, out_hbm.at[pl.ds(base, tpw)])

    return gather_k


def kernel(hidden_states, router_w, w1, w2, w3):
    bsz, seq_len, dim = hidden_states.shape
    x = hidden_states.reshape(-1, dim)

    logits, pm, wm, g32 = _run_router(x, router_w)
    p0 = pm[:, 0]
    p1 = pm[:, 1]
    tile_meta = g32

    x_sorted = _make_scatter()(x, p0, p1)
    y_sorted = _run_mlp(x_sorted, w1, w3, w2, tile_meta)
    final = _make_gather_add()(y_sorted, p0, p1, wm[:, 0], wm[:, 1])
    return (final.reshape(bsz, seq_len, dim), logits)


# speedup vs baseline: 1.0093x; 1.0093x over previous
import functools

import jax
import jax.numpy as jnp
from jax import lax
from jax.experimental import pallas as pl
from jax.experimental.pallas import tpu as pltpu
from jax.experimental.pallas import tpu_sc as plsc

T = 2048
D = 768
FF = 2048
NE = 8
TM = 256
NT = T * 2 // TM + (NE - 1)
XS = NT * TM
TF = 512
NF = FF // TF
CH = 256


def _router_body(x_ref, rw_ref, logits_ref, p0_ref, p1_ref, w0_ref, w1_ref,
                 g_ref):
    x = x_ref[...]
    rw = rw_ref[...]
    logits = lax.dot_general(x, rw, (((1,), (1,)), ((), ())),
                             preferred_element_type=jnp.float32)
    logits_ref[...] = logits

    mx = jnp.max(logits, axis=1, keepdims=True)
    ex = jnp.exp(logits - mx)
    sm = ex / jnp.sum(ex, axis=1, keepdims=True)

    iota_e = lax.broadcasted_iota(jnp.int32, (T, NE), 1).astype(jnp.float32)
    m0 = jnp.max(logits, axis=1, keepdims=True)
    is0 = logits >= m0
    e0 = jnp.min(jnp.where(is0, iota_e, jnp.float32(NE)), axis=1, keepdims=True)
    oh0 = (iota_e == e0).astype(jnp.float32)
    masked = jnp.where(oh0 > 0, -jnp.inf, logits)
    m1 = jnp.max(masked, axis=1, keepdims=True)
    is1 = masked >= m1
    e1 = jnp.min(jnp.where(is1, iota_e, jnp.float32(NE)), axis=1, keepdims=True)
    oh1 = (iota_e == e1).astype(jnp.float32)

    p0v = jnp.sum(sm * oh0, axis=1, keepdims=True)
    p1v = jnp.sum(sm * oh1, axis=1, keepdims=True)
    den = p0v + p1v
    w0_ref[...] = p0v / den
    w1_ref[...] = p1v / den

    s = oh0 + oh1
    r = lax.broadcasted_iota(jnp.int32, (CH, CH), 0)
    c = lax.broadcasted_iota(jnp.int32, (CH, CH), 1)
    tri = (c < r).astype(jnp.float32)
    carry = jnp.zeros((1, NE), jnp.float32)
    chunks = []
    for k in range(T // CH):
        sc = lax.slice_in_dim(s, k * CH, (k + 1) * CH, axis=0)
        cc = lax.dot_general(tri, sc, (((1,), (0,)), ((), ())),
                             preferred_element_type=jnp.float32) + carry
        chunks.append(cc)
        carry = carry + jnp.sum(sc, axis=0, keepdims=True)
    cexc = jnp.concatenate(chunks, axis=0)
    counts = carry

    pc = jnp.ceil(counts / TM) * TM
    rr = lax.broadcasted_iota(jnp.int32, (NE, NE), 0)
    cc2 = lax.broadcasted_iota(jnp.int32, (NE, NE), 1)
    triu = (rr < cc2).astype(jnp.float32)
    pstart = lax.dot_general(pc, triu, (((1,), (0,)), ((), ())),
                             preferred_element_type=jnp.float32)

    rank0 = jnp.sum(cexc * oh0, axis=1, keepdims=True)
    rank1 = jnp.sum((cexc + oh0) * oh1, axis=1, keepdims=True)
    ps0 = jnp.sum(pstart * oh0, axis=1, keepdims=True)
    ps1 = jnp.sum(pstart * oh1, axis=1, keepdims=True)
    p0_ref[...] = (ps0 + rank0).astype(jnp.int32)
    p1_ref[...] = (ps1 + rank1).astype(jnp.int32)

    ti = lax.broadcasted_iota(jnp.int32, (32, 1), 0).astype(jnp.float32) * TM
    tib = jnp.broadcast_to(ti, (32, NE))
    psb = jnp.broadcast_to(pstart, (32, NE))
    nonempty = jnp.broadcast_to(pc > 0, (32, NE))
    cmp = (psb <= tib).astype(jnp.int32)
    g_raw = jnp.sum(cmp, axis=1, keepdims=True) - 1
    iota8 = lax.broadcasted_iota(jnp.int32, (1, NE), 1)
    last_e = jnp.max(jnp.where(counts > 0, iota8, 0), axis=1, keepdims=True)
    ntiles = (jnp.sum(pc, axis=1, keepdims=True) / TM).astype(jnp.int32)
    ntb = jnp.broadcast_to(ntiles, (32, 1))
    row_i = lax.broadcasted_iota(jnp.int32, (32, 1), 0)
    real = (row_i < ntb).astype(jnp.int32)
    started = ((psb <= tib) & nonempty).astype(jnp.int32)
    ordp1 = jnp.sum(started, axis=1, keepdims=True)
    iota8b = jnp.broadcast_to(iota8, (32, NE))
    nxt = jnp.min(jnp.where((psb > tib) & nonempty, iota8b, NE),
                  axis=1, keepdims=True)
    has_next = (nxt < NE).astype(jnp.int32) * real
    first = jnp.sum(((psb == tib) & nonempty).astype(jnp.int32),
                    axis=1, keepdims=True) * real
    g_ref[:, 0:1] = jnp.minimum(g_raw, last_e)
    g_ref[:, 1:2] = (ordp1 - 1) % 2
    g_ref[:, 2:3] = first
    g_ref[:, 3:4] = has_next
    g_ref[:, 4:5] = jnp.where(has_next > 0, nxt, 0)
    g_ref[:, 5:6] = ntb
    g_ref[:, 6:8] = jnp.zeros((32, 2), jnp.int32)


def _run_router(x, router_w):
    return pl.pallas_call(
        _router_body,
        out_shape=(
            jax.ShapeDtypeStruct((T, NE), jnp.float32),
            jax.ShapeDtypeStruct((T, 1), jnp.int32),
            jax.ShapeDtypeStruct((T, 1), jnp.int32),
            jax.ShapeDtypeStruct((T, 1), jnp.float32),
            jax.ShapeDtypeStruct((T, 1), jnp.float32),
            jax.ShapeDtypeStruct((32, 8), jnp.int32),
        ),
    )(x, router_w)


def _mlp_body(m_ref, x_ref, w1_hbm, w3_hbm, w2_hbm, out_ref,
              w1b, w3b, w2b, sems):
    i = pl.program_id(0)
    nt = m_ref[0, 5]
    slot = m_ref[i, 1]
    first = m_ref[i, 2]
    has_next = m_ref[i, 3]
    nextg = m_ref[i, 4]
    pairs = ((w1_hbm, w1b), (w3_hbm, w3b), (w2_hbm, w2b))

    def issue(e, s):
        for k, (hbm, buf) in enumerate(pairs):
            pltpu.make_async_copy(hbm.at[e], buf.at[s], sems.at[k, s]).start()

    def wait_one(k, s):
        hbm, buf = pairs[k]
        pltpu.make_async_copy(hbm.at[0], buf.at[s], sems.at[k, s]).wait()

    @pl.when(i == 0)
    def _():
        issue(m_ref[0, 0], 0)

    @pl.when((first == 1) & (has_next == 1) & (slot == 0))
    def _():
        issue(nextg, 1)

    @pl.when((first == 1) & (has_next == 1) & (slot == 1))
    def _():
        issue(nextg, 0)

    @pl.when((first == 1) & (slot == 0))
    def _():
        for k in range(3):
            wait_one(k, 0)

    @pl.when((first == 1) & (slot == 1))
    def _():
        for k in range(3):
            wait_one(k, 1)

    def compute(s):
        x = x_ref[...].astype(jnp.bfloat16)
        h1 = lax.dot_general(x, w1b[s].astype(jnp.bfloat16),
                             (((1,), (1,)), ((), ())),
                             preferred_element_type=jnp.float32)
        h3 = lax.dot_general(x, w3b[s].astype(jnp.bfloat16),
                             (((1,), (1,)), ((), ())),
                             preferred_element_type=jnp.float32)
        act = (h1 * jax.nn.sigmoid(h1) * h3).astype(jnp.bfloat16)
        out_ref[...] = lax.dot_general(act, w2b[s].astype(jnp.bfloat16),
                                       (((1,), (1,)), ((), ())),
                                       preferred_element_type=jnp.float32)

    @pl.when((i < nt) & (slot == 0))
    def _():
        compute(0)

    @pl.when((i < nt) & (slot == 1))
    def _():
        compute(1)


def _run_mlp(x_sorted, w1, w3, w2, tile_meta):
    grid_spec = pltpu.PrefetchScalarGridSpec(
        num_scalar_prefetch=1,
        grid=(NT,),
        in_specs=[
            pl.BlockSpec((TM, D), lambda i, m: (jnp.minimum(i, m[0, 5] - 1), 0)),
            pl.BlockSpec(memory_space=pl.ANY),
            pl.BlockSpec(memory_space=pl.ANY),
            pl.BlockSpec(memory_space=pl.ANY),
        ],
        out_specs=pl.BlockSpec((TM, D), lambda i, m: (i, 0)),
        scratch_shapes=[
            pltpu.VMEM((2, FF, D), jnp.float32),
            pltpu.VMEM((2, FF, D), jnp.float32),
            pltpu.VMEM((2, D, FF), jnp.float32),
            pltpu.SemaphoreType.DMA((3, 2)),
        ],
    )
    return pl.pallas_call(
        _mlp_body,
        grid_spec=grid_spec,
        out_shape=jax.ShapeDtypeStruct((XS, D), jnp.float32),
        compiler_params=pltpu.CompilerParams(
            dimension_semantics=("arbitrary",)),
    )(tile_meta, x_sorted, w1, w3, w2)


def _make_scatter():
    info = plsc.get_sparse_core_info()
    nw = info.num_cores * info.num_subcores
    tpw = T // nw
    mesh = plsc.VectorSubcoreMesh(core_axis_name="c", subcore_axis_name="s")

    @functools.partial(
        pl.kernel, mesh=mesh,
        out_type=jax.ShapeDtypeStruct((XS, D), jnp.float32),
        scratch_types=[
            pltpu.VMEM((tpw, D), jnp.float32),
            pltpu.VMEM((tpw,), jnp.int32),
            pltpu.SemaphoreType.DMA,
        ],
    )
    def scatter_k(x_hbm, p0_hbm, p1_hbm, xs_hbm, rows_v, idx_v, sem):
        wid = lax.axis_index("s") * info.num_cores + lax.axis_index("c")
        base = wid * tpw
        pltpu.sync_copy(x_hbm.at[pl.ds(base, tpw)], rows_v)
        pltpu.sync_copy(p0_hbm.at[pl.ds(base, tpw)], idx_v)
        pltpu.async_copy(rows_v, xs_hbm.at[idx_v], sem).wait()
        pltpu.sync_copy(p1_hbm.at[pl.ds(base, tpw)], idx_v)
        pltpu.async_copy(rows_v, xs_hbm.at[idx_v], sem).wait()

    return scatter_k


def _make_gather_add():
    info = plsc.get_sparse_core_info()
    nw = info.num_cores * info.num_subcores
    tpw = T // nw
    mesh = plsc.VectorSubcoreMesh(core_axis_name="c", subcore_axis_name="s")

    @functools.partial(
        pl.kernel, mesh=mesh,
        out_type=jax.ShapeDtypeStruct((T, D), jnp.float32),
        scratch_types=[
            pltpu.VMEM((tpw, D), jnp.float32),
            pltpu.VMEM((tpw, D), jnp.float32),
            pltpu.VMEM((tpw,), jnp.int32),
            pltpu.VMEM((tpw,), jnp.float32),
            pltpu.VMEM((tpw,), jnp.float32),
            pltpu.SemaphoreType.DMA,
        ],
    )
    def gather_k(ys_hbm, p0_hbm, p1_hbm, w0_hbm, w1_hbm, out_hbm,
                 r0, r1, idx_v, wv0, wv1, sem):
        wid = lax.axis_index("s") * info.num_cores + lax.axis_index("c")
        base = wid * tpw
        pltpu.sync_copy(p0_hbm.at[pl.ds(base, tpw)], idx_v)
        pltpu.async_copy(ys_hbm.at[idx_v], r0, sem).wait()
        pltpu.sync_copy(p1_hbm.at[pl.ds(base, tpw)], idx_v)
        pltpu.async_copy(ys_hbm.at[idx_v], r1, sem).wait()
        pltpu.sync_copy(w0_hbm.at[pl.ds(base, tpw)], wv0)
        pltpu.sync_copy(w1_hbm.at[pl.ds(base, tpw)], wv1)

        dn = lax.GatherDimensionNumbers(offset_dims=(),
                                        collapsed_slice_dims=(0,),
                                        start_index_map=(0,))

        def row_fma(jt, carry):
            cb = pl.multiple_of((jt // 16) * 16, 8)
            ln = jnp.full((16, 1), jt % 16, jnp.int32)
            ch0 = wv0[pl.ds(cb, 16)]
            ch1 = wv1[pl.ds(cb, 16)]
            b0 = lax.gather(ch0, ln, dn, (1,),
                            mode=lax.GatherScatterMode.PROMISE_IN_BOUNDS)
            b1 = lax.gather(ch1, ln, dn, (1,),
                            mode=lax.GatherScatterMode.PROMISE_IN_BOUNDS)
            for c in range(D // 16):
                sl = pl.ds(c * 16, 16)
                r0[jt, sl] = r0[jt, sl] * b0 + r1[jt, sl] * b1
            return carry

        lax.fori_loop(0, tpw, row_fma, 0)
        pltpu.sync_copy(r0, out_hbm.at[pl.ds(base, tpw)])

    return gather_k


def kernel(hidden_states, router_w, w1, w2, w3):
    bsz, seq_len, dim = hidden_states.shape
    x = hidden_states.reshape(-1, dim)

    logits, p0c, p1c, w0c, w1c, tile_meta = _run_router(x, router_w)
    p0 = p0c.reshape(T)
    p1 = p1c.reshape(T)

    x_sorted = _make_scatter()(x, p0, p1)
    y_sorted = _run_mlp(x_sorted, w1, w3, w2, tile_meta)
    final = _make_gather_add()(y_sorted, p0, p1,
                               w0c.reshape(T), w1c.reshape(T))
    return (final.reshape(bsz, seq_len, dim), logits)


# speedup vs baseline: 1.0094x; 1.0001x over previous
import functools

import jax
import jax.numpy as jnp
from jax import lax
from jax.experimental import pallas as pl
from jax.experimental.pallas import tpu as pltpu
from jax.experimental.pallas import tpu_sc as plsc

T = 2048
D = 768
FF = 2048
NE = 8
TM = 256
NT = T * 2 // TM + (NE - 1)
XS = NT * TM
TF = 512
NF = FF // TF
CH = 256


def _router_body(x_ref, rw_ref, logits_ref, p0_ref, p1_ref, w0_ref, w1_ref,
                 g_ref):
    x = x_ref[...]
    rw = rw_ref[...]
    logits = lax.dot_general(x, rw, (((1,), (1,)), ((), ())),
                             preferred_element_type=jnp.float32)
    logits_ref[...] = logits

    mx = jnp.max(logits, axis=1, keepdims=True)
    ex = jnp.exp(logits - mx)
    sm = ex / jnp.sum(ex, axis=1, keepdims=True)

    iota_e = lax.broadcasted_iota(jnp.int32, (T, NE), 1).astype(jnp.float32)
    m0 = jnp.max(logits, axis=1, keepdims=True)
    is0 = logits >= m0
    e0 = jnp.min(jnp.where(is0, iota_e, jnp.float32(NE)), axis=1, keepdims=True)
    oh0 = (iota_e == e0).astype(jnp.float32)
    masked = jnp.where(oh0 > 0, -jnp.inf, logits)
    m1 = jnp.max(masked, axis=1, keepdims=True)
    is1 = masked >= m1
    e1 = jnp.min(jnp.where(is1, iota_e, jnp.float32(NE)), axis=1, keepdims=True)
    oh1 = (iota_e == e1).astype(jnp.float32)

    p0v = jnp.sum(sm * oh0, axis=1, keepdims=True)
    p1v = jnp.sum(sm * oh1, axis=1, keepdims=True)
    den = p0v + p1v
    w0_ref[...] = p0v / den
    w1_ref[...] = p1v / den

    s = oh0 + oh1
    r = lax.broadcasted_iota(jnp.int32, (CH, CH), 0)
    c = lax.broadcasted_iota(jnp.int32, (CH, CH), 1)
    tri = (c < r).astype(jnp.float32)
    carry = jnp.zeros((1, NE), jnp.float32)
    chunks = []
    for k in range(T // CH):
        sc = lax.slice_in_dim(s, k * CH, (k + 1) * CH, axis=0)
        cc = lax.dot_general(tri, sc, (((1,), (0,)), ((), ())),
                             preferred_element_type=jnp.float32) + carry
        chunks.append(cc)
        carry = carry + jnp.sum(sc, axis=0, keepdims=True)
    cexc = jnp.concatenate(chunks, axis=0)
    counts = carry

    pc = jnp.ceil(counts / TM) * TM
    rr = lax.broadcasted_iota(jnp.int32, (NE, NE), 0)
    cc2 = lax.broadcasted_iota(jnp.int32, (NE, NE), 1)
    triu = (rr < cc2).astype(jnp.float32)
    pstart = lax.dot_general(pc, triu, (((1,), (0,)), ((), ())),
                             preferred_element_type=jnp.float32)

    rank0 = jnp.sum(cexc * oh0, axis=1, keepdims=True)
    rank1 = jnp.sum((cexc + oh0) * oh1, axis=1, keepdims=True)
    ps0 = jnp.sum(pstart * oh0, axis=1, keepdims=True)
    ps1 = jnp.sum(pstart * oh1, axis=1, keepdims=True)
    p0_ref[...] = (ps0 + rank0).astype(jnp.int32)
    p1_ref[...] = (ps1 + rank1).astype(jnp.int32)

    ti = lax.broadcasted_iota(jnp.int32, (32, 1), 0).astype(jnp.float32) * TM
    tib = jnp.broadcast_to(ti, (32, NE))
    psb = jnp.broadcast_to(pstart, (32, NE))
    nonempty = jnp.broadcast_to(pc > 0, (32, NE))
    cmp = (psb <= tib).astype(jnp.int32)
    g_raw = jnp.sum(cmp, axis=1, keepdims=True) - 1
    iota8 = lax.broadcasted_iota(jnp.int32, (1, NE), 1)
    last_e = jnp.max(jnp.where(counts > 0, iota8, 0), axis=1, keepdims=True)
    ntiles = (jnp.sum(pc, axis=1, keepdims=True) / TM).astype(jnp.int32)
    ntb = jnp.broadcast_to(ntiles, (32, 1))
    row_i = lax.broadcasted_iota(jnp.int32, (32, 1), 0)
    real = (row_i < ntb).astype(jnp.int32)
    started = ((psb <= tib) & nonempty).astype(jnp.int32)
    ordp1 = jnp.sum(started, axis=1, keepdims=True)
    iota8b = jnp.broadcast_to(iota8, (32, NE))
    nxt = jnp.min(jnp.where((psb > tib) & nonempty, iota8b, NE),
                  axis=1, keepdims=True)
    has_next = (nxt < NE).astype(jnp.int32) * real
    first = jnp.sum(((psb == tib) & nonempty).astype(jnp.int32),
                    axis=1, keepdims=True) * real
    g_ref[:, 0:1] = jnp.minimum(g_raw, last_e)
    g_ref[:, 1:2] = (ordp1 - 1) % 2
    g_ref[:, 2:3] = first
    g_ref[:, 3:4] = has_next
    g_ref[:, 4:5] = jnp.where(has_next > 0, nxt, 0)
    g_ref[:, 5:6] = ntb
    g_ref[:, 6:8] = jnp.zeros((32, 2), jnp.int32)


def _run_router(x, router_w):
    return pl.pallas_call(
        _router_body,
        out_shape=(
            jax.ShapeDtypeStruct((T, NE), jnp.float32),
            jax.ShapeDtypeStruct((T, 1), jnp.int32),
            jax.ShapeDtypeStruct((T, 1), jnp.int32),
            jax.ShapeDtypeStruct((T, 1), jnp.float32),
            jax.ShapeDtypeStruct((T, 1), jnp.float32),
            jax.ShapeDtypeStruct((32, 8), jnp.int32),
        ),
    )(x, router_w)


def _mlp_body(m_ref, x_ref, w1_hbm, w3_hbm, w2_hbm, out_ref,
              w1b, w3b, w2b, sems):
    i = pl.program_id(0)
    nt = m_ref[0, 5]
    slot = m_ref[i, 1]
    first = m_ref[i, 2]
    has_next = m_ref[i, 3]
    nextg = m_ref[i, 4]
    pairs = ((w1_hbm, w1b), (w3_hbm, w3b), (w2_hbm, w2b))

    def issue(e, s):
        for k, (hbm, buf) in enumerate(pairs):
            pltpu.make_async_copy(hbm.at[e], buf.at[s], sems.at[k, s]).start()

    def wait_one(k, s):
        hbm, buf = pairs[k]
        pltpu.make_async_copy(hbm.at[0], buf.at[s], sems.at[k, s]).wait()

    @pl.when(i == 0)
    def _():
        issue(m_ref[0, 0], 0)

    @pl.when((first == 1) & (has_next == 1) & (slot == 0))
    def _():
        issue(nextg, 1)

    @pl.when((first == 1) & (has_next == 1) & (slot == 1))
    def _():
        issue(nextg, 0)

    @pl.when((first == 1) & (slot == 0))
    def _():
        for k in range(3):
            wait_one(k, 0)

    @pl.when((first == 1) & (slot == 1))
    def _():
        for k in range(3):
            wait_one(k, 1)

    def compute(s):
        x = x_ref[...].astype(jnp.bfloat16)
        h1 = lax.dot_general(x, w1b[s].astype(jnp.bfloat16),
                             (((1,), (1,)), ((), ())),
                             preferred_element_type=jnp.float32)
        h3 = lax.dot_general(x, w3b[s].astype(jnp.bfloat16),
                             (((1,), (1,)), ((), ())),
                             preferred_element_type=jnp.float32)
        act = (h1 * jax.nn.sigmoid(h1) * h3).astype(jnp.bfloat16)
        out_ref[...] = lax.dot_general(act, w2b[s].astype(jnp.bfloat16),
                                       (((1,), (1,)), ((), ())),
                                       preferred_element_type=jnp.float32)

    @pl.when((i < nt) & (slot == 0))
    def _():
        compute(0)

    @pl.when((i < nt) & (slot == 1))
    def _():
        compute(1)


def _run_mlp(x_sorted, w1, w3, w2, tile_meta):
    grid_spec = pltpu.PrefetchScalarGridSpec(
        num_scalar_prefetch=1,
        grid=(NT,),
        in_specs=[
            pl.BlockSpec((TM, D), lambda i, m: (jnp.minimum(i, m[0, 5] - 1), 0)),
            pl.BlockSpec(memory_space=pl.ANY),
            pl.BlockSpec(memory_space=pl.ANY),
            pl.BlockSpec(memory_space=pl.ANY),
        ],
        out_specs=pl.BlockSpec((TM, D), lambda i, m: (i, 0)),
        scratch_shapes=[
            pltpu.VMEM((2, FF, D), jnp.float32),
            pltpu.VMEM((2, FF, D), jnp.float32),
            pltpu.VMEM((2, D, FF), jnp.float32),
            pltpu.SemaphoreType.DMA((3, 2)),
        ],
    )
    return pl.pallas_call(
        _mlp_body,
        grid_spec=grid_spec,
        out_shape=jax.ShapeDtypeStruct((XS, D), jnp.float32),
        compiler_params=pltpu.CompilerParams(
            dimension_semantics=("arbitrary",)),
    )(tile_meta, x_sorted, w1, w3, w2)


def _make_scatter():
    info = plsc.get_sparse_core_info()
    nw = info.num_cores * info.num_subcores
    tpw = T // nw
    mesh = plsc.VectorSubcoreMesh(core_axis_name="c", subcore_axis_name="s")

    @functools.partial(
        pl.kernel, mesh=mesh,
        out_type=jax.ShapeDtypeStruct((XS, D), jnp.float32),
        scratch_types=[
            pltpu.VMEM((tpw, D), jnp.float32),
            pltpu.VMEM((tpw,), jnp.int32),
            pltpu.SemaphoreType.DMA,
        ],
    )
    def scatter_k(x_hbm, p0_hbm, p1_hbm, xs_hbm, rows_v, idx_v, sem):
        wid = lax.axis_index("s") * info.num_cores + lax.axis_index("c")
        base = wid * tpw
        pltpu.sync_copy(x_hbm.at[pl.ds(base, tpw)], rows_v)
        pltpu.sync_copy(p0_hbm.at[pl.ds(base, tpw)], idx_v)
        pltpu.async_copy(rows_v, xs_hbm.at[idx_v], sem).wait()
        pltpu.sync_copy(p1_hbm.at[pl.ds(base, tpw)], idx_v)
        pltpu.async_copy(rows_v, xs_hbm.at[idx_v], sem).wait()

    return scatter_k


def _make_gather_add():
    info = plsc.get_sparse_core_info()
    nw = info.num_cores * info.num_subcores
    tpw = T // nw
    mesh = plsc.VectorSubcoreMesh(core_axis_name="c", subcore_axis_name="s")
    hp = tpw // 2

    @functools.partial(
        pl.kernel, mesh=mesh,
        out_type=jax.ShapeDtypeStruct((T, D), jnp.float32),
        scratch_types=[
            pltpu.VMEM((tpw, D), jnp.float32),
            pltpu.VMEM((tpw, D), jnp.float32),
            pltpu.VMEM((tpw,), jnp.int32),
            pltpu.VMEM((tpw,), jnp.int32),
            pltpu.VMEM((tpw,), jnp.float32),
            pltpu.VMEM((tpw,), jnp.float32),
            pltpu.SemaphoreType.DMA,
            pltpu.SemaphoreType.DMA,
        ],
    )
    def gather_k(ys_hbm, p0_hbm, p1_hbm, w0_hbm, w1_hbm, out_hbm,
                 r0, r1, idx0_v, idx1_v, wv0, wv1, sem_a, sem_b):
        wid = lax.axis_index("s") * info.num_cores + lax.axis_index("c")
        base = wid * tpw
        pltpu.sync_copy(p0_hbm.at[pl.ds(base, tpw)], idx0_v)
        pltpu.sync_copy(p1_hbm.at[pl.ds(base, tpw)], idx1_v)
        pltpu.sync_copy(w0_hbm.at[pl.ds(base, tpw)], wv0)
        pltpu.sync_copy(w1_hbm.at[pl.ds(base, tpw)], wv1)
        ha = pl.ds(0, hp)
        hb = pl.ds(hp, hp)
        ca0 = pltpu.async_copy(ys_hbm.at[idx0_v.at[ha]], r0.at[ha], sem_a)
        ca1 = pltpu.async_copy(ys_hbm.at[idx1_v.at[ha]], r1.at[ha], sem_a)
        cb0 = pltpu.async_copy(ys_hbm.at[idx0_v.at[hb]], r0.at[hb], sem_b)
        cb1 = pltpu.async_copy(ys_hbm.at[idx1_v.at[hb]], r1.at[hb], sem_b)

        dn = lax.GatherDimensionNumbers(offset_dims=(),
                                        collapsed_slice_dims=(0,),
                                        start_index_map=(0,))

        def row_fma(jt, carry):
            cb = pl.multiple_of((jt // 16) * 16, 8)
            ln = jnp.full((16, 1), jt % 16, jnp.int32)
            ch0 = wv0[pl.ds(cb, 16)]
            ch1 = wv1[pl.ds(cb, 16)]
            b0 = lax.gather(ch0, ln, dn, (1,),
                            mode=lax.GatherScatterMode.PROMISE_IN_BOUNDS)
            b1 = lax.gather(ch1, ln, dn, (1,),
                            mode=lax.GatherScatterMode.PROMISE_IN_BOUNDS)
            for c in range(D // 16):
                sl = pl.ds(c * 16, 16)
                r0[jt, sl] = r0[jt, sl] * b0 + r1[jt, sl] * b1
            return carry

        ca0.wait()
        ca1.wait()
        lax.fori_loop(0, hp, row_fma, 0)
        cb0.wait()
        cb1.wait()
        lax.fori_loop(hp, tpw, row_fma, 0)
        pltpu.sync_copy(r0, out_hbm.at[pl.ds(base, tpw)])

    return gather_k


def kernel(hidden_states, router_w, w1, w2, w3):
    bsz, seq_len, dim = hidden_states.shape
    x = hidden_states.reshape(-1, dim)

    logits, p0c, p1c, w0c, w1c, tile_meta = _run_router(x, router_w)
    p0 = p0c.reshape(T)
    p1 = p1c.reshape(T)

    x_sorted = _make_scatter()(x, p0, p1)
    y_sorted = _run_mlp(x_sorted, w1, w3, w2, tile_meta)
    final = _make_gather_add()(y_sorted, p0, p1,
                               w0c.reshape(T), w1c.reshape(T))
    return (final.reshape(bsz, seq_len, dim), logits)
